# baseline (speedup 1.0000x reference)
import functools

import jax
import jax.numpy as jnp
from jax import lax
from jax.experimental import pallas as pl
from jax.experimental.pallas import tpu as pltpu
from jax.experimental.pallas import tpu_sc as plsc

N = 10000
E = 320000
D = 128
DH = D // 2

NC = 2
NS = 16
EPT = E // NS
B = 125
NCHUNK = EPT // B
NSLOT = 2
NR = NCHUNK // NSLOT
NP = 10240
RPT = NP // NS
XCH = 128
NX = RPT // XCH

ROWS_TC = 1000
GRID_TC = N // ROWS_TC


def _seg_sum_sc(t_lo, t_hi, src3, dst3, zrows, zcnt, ones):
    mesh = plsc.VectorSubcoreMesh(core_axis_name="c", subcore_axis_name="s")

    @functools.partial(
        pl.kernel,
        out_type=(
            jax.ShapeDtypeStruct((NC, NP, DH), jnp.float32),
            jax.ShapeDtypeStruct((NC, NP), jnp.float32),
        ),
        mesh=mesh,
        scratch_types=(
            pltpu.VMEM((NCHUNK, B), jnp.int32),
            pltpu.VMEM((NCHUNK, B), jnp.int32),
            [pltpu.VMEM((B, DH), jnp.float32) for _ in range(NSLOT)],
            pltpu.VMEM((XCH, DH), jnp.float32),
            pltpu.VMEM((B,), jnp.float32),
            pltpu.VMEM((NP,), jnp.float32),
            pltpu.VMEM_SHARED((NP, DH), jnp.float32),
            pltpu.VMEM_SHARED((NP,), jnp.float32),
            [pltpu.SemaphoreType.DMA for _ in range(NSLOT)],
            [pltpu.SemaphoreType.DMA for _ in range(NSLOT)],
        ),
        compiler_params=pltpu.CompilerParams(use_tc_tiling_on_sc=False),
    )
    def k(tlo_hbm, thi_hbm, src_hbm, dst_hbm, zrows_hbm, zcnt_hbm, ones_hbm,
          agg_out, cnt_out,
          src_v, dst_v, bufs, xfer, ones_v, cbuf,
          agg_sh, cnt_sh, gsems, ssems):
        c = lax.axis_index("c")
        s = lax.axis_index("s")
        base = s * RPT

        pltpu.sync_copy(src_hbm.at[s], src_v)
        pltpu.sync_copy(dst_hbm.at[s], dst_v)
        pltpu.sync_copy(ones_hbm, ones_v)

        pltpu.sync_copy(zrows_hbm, xfer)
        for kx in range(NX):
            pltpu.sync_copy(xfer, agg_sh.at[pl.ds(base + kx * XCH, XCH)])

        @pl.when(s == 0)
        def _zero_cnt():
            pltpu.sync_copy(zcnt_hbm, cbuf)
            pltpu.sync_copy(cbuf, cnt_sh)

        plsc.subcore_barrier()

        def fire_gather(j, i):
            @pl.when(c == 0)
            def _lo():
                pltpu.async_copy(tlo_hbm.at[src_v.at[j]], bufs[i], gsems[i])

            @pl.when(c == 1)
            def _hi():
                pltpu.async_copy(thi_hbm.at[src_v.at[j]], bufs[i], gsems[i])

        def wait_gather(i):
            pltpu.make_async_copy(tlo_hbm.at[src_v.at[0]], bufs[i],
                                  gsems[i]).wait()

        def fire_scatter(j, i):
            pltpu.async_copy(bufs[i], agg_sh.at[dst_v.at[j]], ssems[i],
                             add=True)
            pltpu.async_copy(ones_v, cnt_sh.at[dst_v.at[j]], ssems[i],
                             add=True)

        def wait_scatter(i):
            pltpu.make_async_copy(bufs[i], agg_sh.at[dst_v.at[0]],
                                  ssems[i]).wait()
            pltpu.make_async_copy(ones_v, cnt_sh.at[dst_v.at[0]],
                                  ssems[i]).wait()

        for i in range(NSLOT):
            fire_gather(i, i)

        @pl.loop(0, NR - 1)
        def _main(r):
            jj = r * NSLOT
            for i in range(NSLOT):
                wait_gather(i)
                fire_scatter(jj + i, i)
            for i in range(NSLOT):
                wait_scatter(i)
                fire_gather(jj + NSLOT + i, i)

        jj = (NR - 1) * NSLOT
        for i in range(NSLOT):
            wait_gather(i)
            fire_scatter(jj + i, i)
        for i in range(NSLOT):
            wait_scatter(i)

        plsc.subcore_barrier()

        for kx in range(NX):
            pltpu.sync_copy(agg_sh.at[pl.ds(base + kx * XCH, XCH)], xfer)
            pltpu.sync_copy(xfer, agg_out.at[c, pl.ds(base + kx * XCH, XCH)])

        @pl.when(s == 0)
        def _export_cnt():
            pltpu.sync_copy(cnt_sh, cbuf)
            pltpu.sync_copy(cbuf, cnt_out.at[c])

    return k(t_lo, t_hi, src3, dst3, zrows, zcnt, ones)


def _r_body(x_ref, w_ref, b_ref, r_ref):
    r_ref[...] = (
        jnp.dot(x_ref[...], w_ref[...], preferred_element_type=jnp.float32)
        + b_ref[...]
    )


def _mid_body(agg_ref, cnt_ref, r_ref, wl_ref, h_ref):
    mean = jnp.concatenate([agg_ref[0], agg_ref[1]], axis=-1)
    mean = mean / jnp.maximum(cnt_ref[0], 1.0)
    h_ref[...] = jnp.maximum(
        jnp.dot(mean, wl_ref[...], preferred_element_type=jnp.float32)
        + r_ref[...],
        0.0,
    )


def _fin_body(agg_ref, cnt_ref, r_ref, wl_ref, out_ref):
    mean = jnp.concatenate([agg_ref[0], agg_ref[1]], axis=-1)
    mean = mean / jnp.maximum(cnt_ref[0], 1.0)
    out_ref[...] = (
        jnp.dot(mean, wl_ref[...], preferred_element_type=jnp.float32)
        + r_ref[...]
    )


def _rows_spec():
    return pl.BlockSpec((ROWS_TC, D), lambda i: (i, 0))


def _w_spec():
    return pl.BlockSpec((D, D), lambda i: (0, 0))


def _b_spec():
    return pl.BlockSpec((1, D), lambda i: (0, 0))


def _agg_spec():
    return pl.BlockSpec((NC, ROWS_TC, DH), lambda i: (0, i, 0))


def _cnt_spec():
    return pl.BlockSpec((NC, ROWS_TC, 1), lambda i: (0, i, 0))


def _r_kernel(x, W, b):
    return pl.pallas_call(
        _r_body,
        grid=(GRID_TC,),
        in_specs=[_rows_spec(), _w_spec(), _b_spec()],
        out_specs=_rows_spec(),
        out_shape=jax.ShapeDtypeStruct((N, D), jnp.float32),
    )(x, W, b.reshape(1, D))


def _mid(agg, cnt, r, Wl):
    return pl.pallas_call(
        _mid_body,
        grid=(GRID_TC,),
        in_specs=[_agg_spec(), _cnt_spec(), _rows_spec(), _w_spec()],
        out_specs=_rows_spec(),
        out_shape=jax.ShapeDtypeStruct((N, D), jnp.float32),
    )(agg, cnt.reshape(NC, NP, 1), r, Wl)


def _fin(agg, cnt, r, Wl):
    return pl.pallas_call(
        _fin_body,
        grid=(GRID_TC,),
        in_specs=[_agg_spec(), _cnt_spec(), _rows_spec(), _w_spec()],
        out_specs=_rows_spec(),
        out_shape=jax.ShapeDtypeStruct((N, D), jnp.float32),
    )(agg, cnt.reshape(NC, NP, 1), r, Wl)


def kernel(x, edge_index_list, Wl0, Wr0, b0, Wl1, Wr1, b1):
    src0 = edge_index_list[0, 0].reshape(NS, NCHUNK, B)
    dst0 = edge_index_list[0, 1].reshape(NS, NCHUNK, B)
    src1 = edge_index_list[1, 0].reshape(NS, NCHUNK, B)
    dst1 = edge_index_list[1, 1].reshape(NS, NCHUNK, B)

    zrows = jnp.zeros((XCH, DH), jnp.float32)
    zcnt = jnp.zeros((NP,), jnp.float32)
    ones = jnp.ones((B,), jnp.float32)

    agg0, cnt0 = _seg_sum_sc(x[:, :DH], x[:, DH:], src0, dst0,
                             zrows, zcnt, ones)
    r0 = _r_kernel(x, Wr0, b0)
    h = _mid(agg0, cnt0, r0, Wl0)

    agg1, cnt1 = _seg_sum_sc(h[:, :DH], h[:, DH:], src1, dst1,
                             zrows, zcnt, ones)
    r1 = _r_kernel(h, Wr1, b1)
    return _fin(agg1, cnt1, r1, Wl1)


# speedup vs baseline: 1.1898x; 1.1898x over previous
import functools

import jax
import jax.numpy as jnp
from jax import lax
from jax.experimental import pallas as pl
from jax.experimental.pallas import tpu as pltpu
from jax.experimental.pallas import tpu_sc as plsc

N = 10000
E = 320000
D = 128
DH = D // 2

NC = 2
NS = 16
EPT = E // NS
B = 125
NCHUNK = EPT // B
NSLOT = 2
NR = NCHUNK // NSLOT
NP = 10240
RPT = NP // NS
XCH = 128
NX = RPT // XCH

ROWS_TC = 1000
GRID_TC = N // ROWS_TC


def _seg_sum_sc(t_lo, t_hi, src3, dst3, zrows, zcnt, ones):
    mesh = plsc.VectorSubcoreMesh(core_axis_name="c", subcore_axis_name="s")

    @functools.partial(
        pl.kernel,
        out_type=(
            jax.ShapeDtypeStruct((NC, NP, DH), jnp.float32),
            jax.ShapeDtypeStruct((NC, NP), jnp.float32),
        ),
        mesh=mesh,
        scratch_types=(
            pltpu.VMEM((NCHUNK, B), jnp.int32),
            pltpu.VMEM((NCHUNK, B), jnp.int32),
            [pltpu.VMEM((B, DH), jnp.float32) for _ in range(NSLOT)],
            pltpu.VMEM((XCH, DH), jnp.float32),
            pltpu.VMEM((B,), jnp.float32),
            pltpu.VMEM((NP,), jnp.float32),
            pltpu.VMEM_SHARED((NP, DH), jnp.float32),
            pltpu.VMEM_SHARED((NP,), jnp.float32),
            [pltpu.SemaphoreType.DMA for _ in range(NSLOT)],
            [pltpu.SemaphoreType.DMA for _ in range(NSLOT)],
        ),
        compiler_params=pltpu.CompilerParams(use_tc_tiling_on_sc=False),
    )
    def k(tlo_hbm, thi_hbm, src_hbm, dst_hbm, zrows_hbm, zcnt_hbm, ones_hbm,
          agg_out, cnt_out,
          src_v, dst_v, bufs, xfer, ones_v, cbuf,
          agg_sh, cnt_sh, gsems, ssems):
        c = lax.axis_index("c")
        s = lax.axis_index("s")
        base = s * RPT

        pltpu.sync_copy(src_hbm.at[s], src_v)
        pltpu.sync_copy(dst_hbm.at[s], dst_v)
        pltpu.sync_copy(ones_hbm, ones_v)

        pltpu.sync_copy(zrows_hbm, xfer)
        for kx in range(NX):
            pltpu.sync_copy(xfer, agg_sh.at[pl.ds(base + kx * XCH, XCH)])

        @pl.when(s == 0)
        def _zero_cnt():
            pltpu.sync_copy(zcnt_hbm, cbuf)
            pltpu.sync_copy(cbuf, cnt_sh)

        plsc.subcore_barrier()

        def fire_gather(j, i):
            @pl.when(c == 0)
            def _lo():
                pltpu.async_copy(tlo_hbm.at[src_v.at[j]], bufs[i], gsems[i])

            @pl.when(c == 1)
            def _hi():
                pltpu.async_copy(thi_hbm.at[src_v.at[j]], bufs[i], gsems[i])

        def wait_gather(i):
            pltpu.make_async_copy(tlo_hbm.at[src_v.at[0]], bufs[i],
                                  gsems[i]).wait()

        def do_scatter(j, i):
            pltpu.async_copy(ones_v, cnt_sh.at[dst_v.at[j]], ssems[i],
                             add=True)
            pltpu.sync_copy(bufs[i], agg_sh.at[dst_v.at[j]], add=True)

        def wait_cnt(i):
            pltpu.make_async_copy(ones_v, cnt_sh.at[dst_v.at[0]],
                                  ssems[i]).wait()

        for i in range(NSLOT):
            fire_gather(i, i)

        @pl.loop(0, NR - 1)
        def _main(r):
            jj = r * NSLOT
            for i in range(NSLOT):
                wait_gather(i)
                do_scatter(jj + i, i)
                wait_cnt(i)
                fire_gather(jj + NSLOT + i, i)

        jj = (NR - 1) * NSLOT
        for i in range(NSLOT):
            wait_gather(i)
            do_scatter(jj + i, i)
            wait_cnt(i)

        plsc.subcore_barrier()

        for kx in range(NX):
            pltpu.sync_copy(agg_sh.at[pl.ds(base + kx * XCH, XCH)], xfer)
            pltpu.sync_copy(xfer, agg_out.at[c, pl.ds(base + kx * XCH, XCH)])

        @pl.when(s == 0)
        def _export_cnt():
            pltpu.sync_copy(cnt_sh, cbuf)
            pltpu.sync_copy(cbuf, cnt_out.at[c])

    return k(t_lo, t_hi, src3, dst3, zrows, zcnt, ones)


def _r_body(x_ref, w_ref, b_ref, r_ref):
    r_ref[...] = (
        jnp.dot(x_ref[...], w_ref[...], preferred_element_type=jnp.float32)
        + b_ref[...]
    )


def _mid_body(agg_ref, cnt_ref, r_ref, wl_ref, h_ref):
    mean = jnp.concatenate([agg_ref[0], agg_ref[1]], axis=-1)
    mean = mean / jnp.maximum(cnt_ref[0], 1.0)
    h_ref[...] = jnp.maximum(
        jnp.dot(mean, wl_ref[...], preferred_element_type=jnp.float32)
        + r_ref[...],
        0.0,
    )


def _fin_body(agg_ref, cnt_ref, r_ref, wl_ref, out_ref):
    mean = jnp.concatenate([agg_ref[0], agg_ref[1]], axis=-1)
    mean = mean / jnp.maximum(cnt_ref[0], 1.0)
    out_ref[...] = (
        jnp.dot(mean, wl_ref[...], preferred_element_type=jnp.float32)
        + r_ref[...]
    )


def _rows_spec():
    return pl.BlockSpec((ROWS_TC, D), lambda i: (i, 0))


def _w_spec():
    return pl.BlockSpec((D, D), lambda i: (0, 0))


def _b_spec():
    return pl.BlockSpec((1, D), lambda i: (0, 0))


def _agg_spec():
    return pl.BlockSpec((NC, ROWS_TC, DH), lambda i: (0, i, 0))


def _cnt_spec():
    return pl.BlockSpec((NC, ROWS_TC, 1), lambda i: (0, i, 0))


def _r_kernel(x, W, b):
    return pl.pallas_call(
        _r_body,
        grid=(GRID_TC,),
        in_specs=[_rows_spec(), _w_spec(), _b_spec()],
        out_specs=_rows_spec(),
        out_shape=jax.ShapeDtypeStruct((N, D), jnp.float32),
    )(x, W, b.reshape(1, D))


def _mid(agg, cnt, r, Wl):
    return pl.pallas_call(
        _mid_body,
        grid=(GRID_TC,),
        in_specs=[_agg_spec(), _cnt_spec(), _rows_spec(), _w_spec()],
        out_specs=_rows_spec(),
        out_shape=jax.ShapeDtypeStruct((N, D), jnp.float32),
    )(agg, cnt.reshape(NC, NP, 1), r, Wl)


def _fin(agg, cnt, r, Wl):
    return pl.pallas_call(
        _fin_body,
        grid=(GRID_TC,),
        in_specs=[_agg_spec(), _cnt_spec(), _rows_spec(), _w_spec()],
        out_specs=_rows_spec(),
        out_shape=jax.ShapeDtypeStruct((N, D), jnp.float32),
    )(agg, cnt.reshape(NC, NP, 1), r, Wl)


def kernel(x, edge_index_list, Wl0, Wr0, b0, Wl1, Wr1, b1):
    src0 = edge_index_list[0, 0].reshape(NS, NCHUNK, B)
    dst0 = edge_index_list[0, 1].reshape(NS, NCHUNK, B)
    src1 = edge_index_list[1, 0].reshape(NS, NCHUNK, B)
    dst1 = edge_index_list[1, 1].reshape(NS, NCHUNK, B)

    zrows = jnp.zeros((XCH, DH), jnp.float32)
    zcnt = jnp.zeros((NP,), jnp.float32)
    ones = jnp.ones((B,), jnp.float32)

    agg0, cnt0 = _seg_sum_sc(x[:, :DH], x[:, DH:], src0, dst0,
                             zrows, zcnt, ones)
    r0 = _r_kernel(x, Wr0, b0)
    h = _mid(agg0, cnt0, r0, Wl0)

    agg1, cnt1 = _seg_sum_sc(h[:, :DH], h[:, DH:], src1, dst1,
                             zrows, zcnt, ones)
    r1 = _r_kernel(h, Wr1, b1)
    return _fin(agg1, cnt1, r1, Wl1)


# speedup vs baseline: 1.2555x; 1.0552x over previous
import functools

import jax
import jax.numpy as jnp
from jax import lax
from jax.experimental import pallas as pl
from jax.experimental.pallas import tpu as pltpu
from jax.experimental.pallas import tpu_sc as plsc

N = 10000
E = 320000
D = 128
DH = D // 2

NC = 2
NS = 16
EPT = E // NS
B = 125
NCHUNK = EPT // B
NSLOT = 2
NR = NCHUNK // NSLOT
NP = 10240
RPT = NP // NS
XCH = 128
NX = RPT // XCH

ROWS_TC = 1000
GRID_TC = N // ROWS_TC


def _seg_sum_sc(t_lo, t_hi, src3, dst3, zrows, zcnt, ones):
    mesh = plsc.VectorSubcoreMesh(core_axis_name="c", subcore_axis_name="s")

    @functools.partial(
        pl.kernel,
        out_type=jax.ShapeDtypeStruct((NC, NP, DH), jnp.float32),
        mesh=mesh,
        scratch_types=(
            pltpu.VMEM((NCHUNK, B), jnp.int32),
            pltpu.VMEM((NCHUNK, B), jnp.int32),
            [pltpu.VMEM((B, DH), jnp.float32) for _ in range(NSLOT)],
            pltpu.VMEM((XCH, DH), jnp.float32),
            pltpu.VMEM((B,), jnp.float32),
            pltpu.VMEM((RPT,), jnp.float32),
            pltpu.VMEM_SHARED((NP, DH), jnp.float32),
            pltpu.VMEM_SHARED((NP,), jnp.float32),
            [pltpu.SemaphoreType.DMA for _ in range(NSLOT)],
            [pltpu.SemaphoreType.DMA for _ in range(NSLOT)],
        ),
        compiler_params=pltpu.CompilerParams(use_tc_tiling_on_sc=False),
    )
    def k(tlo_hbm, thi_hbm, src_hbm, dst_hbm, zrows_hbm, zcnt_hbm, ones_hbm,
          mean_out,
          src_v, dst_v, bufs, xfer, ones_v, cbuf,
          agg_sh, cnt_sh, gsems, ssems):
        c = lax.axis_index("c")
        s = lax.axis_index("s")
        base = s * RPT

        pltpu.sync_copy(src_hbm.at[s], src_v)
        pltpu.sync_copy(dst_hbm.at[s], dst_v)
        pltpu.sync_copy(ones_hbm, ones_v)

        pltpu.sync_copy(zrows_hbm, xfer)
        for kx in range(NX):
            pltpu.sync_copy(xfer, agg_sh.at[pl.ds(base + kx * XCH, XCH)])

        @pl.when(s == 0)
        def _zero_cnt():
            pltpu.sync_copy(zcnt_hbm, cnt_sh)

        plsc.subcore_barrier()

        def fire_gather(j, i):
            @pl.when(c == 0)
            def _lo():
                pltpu.async_copy(tlo_hbm.at[src_v.at[j]], bufs[i], gsems[i])

            @pl.when(c == 1)
            def _hi():
                pltpu.async_copy(thi_hbm.at[src_v.at[j]], bufs[i], gsems[i])

        def wait_gather(i):
            pltpu.make_async_copy(tlo_hbm.at[src_v.at[0]], bufs[i],
                                  gsems[i]).wait()

        def do_scatter(j, i):
            pltpu.async_copy(ones_v, cnt_sh.at[dst_v.at[j]], ssems[i],
                             add=True)
            pltpu.sync_copy(bufs[i], agg_sh.at[dst_v.at[j]], add=True)

        def wait_cnt(i):
            pltpu.make_async_copy(ones_v, cnt_sh.at[dst_v.at[0]],
                                  ssems[i]).wait()

        for i in range(NSLOT):
            fire_gather(i, i)

        @pl.loop(0, NR - 1)
        def _main(r):
            jj = r * NSLOT
            for i in range(NSLOT):
                wait_gather(i)
                do_scatter(jj + i, i)
                wait_cnt(i)
                fire_gather(jj + NSLOT + i, i)

        jj = (NR - 1) * NSLOT
        for i in range(NSLOT):
            wait_gather(i)
            do_scatter(jj + i, i)
            wait_cnt(i)

        plsc.subcore_barrier()

        pltpu.sync_copy(cnt_sh.at[pl.ds(base, RPT)], cbuf)
        for kx in range(NX):
            pltpu.sync_copy(agg_sh.at[pl.ds(base + kx * XCH, XCH)], xfer)

            @pl.loop(0, XCH // 16)
            def _rows(g):
                inv = 1.0 / jnp.maximum(
                    cbuf[pl.ds(kx * XCH + g * 16, 16)], 1.0)
                for l in range(16):
                    row = g * 16 + l
                    for col in range(DH // 16):
                        sl = pl.ds(col * 16, 16)
                        xfer[row, sl] = xfer[row, sl] * inv[l]

            pltpu.sync_copy(xfer, mean_out.at[c, pl.ds(base + kx * XCH, XCH)])

    return k(t_lo, t_hi, src3, dst3, zrows, zcnt, ones)


def _r_body(x_ref, w_ref, b_ref, r_ref):
    r_ref[...] = (
        jnp.dot(x_ref[...], w_ref[...], preferred_element_type=jnp.float32)
        + b_ref[...]
    )


def _mid_body(agg_ref, r_ref, wl_ref, h_ref):
    mean = jnp.concatenate([agg_ref[0], agg_ref[1]], axis=-1)
    h_ref[...] = jnp.maximum(
        jnp.dot(mean, wl_ref[...], preferred_element_type=jnp.float32)
        + r_ref[...],
        0.0,
    )


def _fin_body(agg_ref, r_ref, wl_ref, out_ref):
    mean = jnp.concatenate([agg_ref[0], agg_ref[1]], axis=-1)
    out_ref[...] = (
        jnp.dot(mean, wl_ref[...], preferred_element_type=jnp.float32)
        + r_ref[...]
    )


def _rows_spec():
    return pl.BlockSpec((ROWS_TC, D), lambda i: (i, 0))


def _w_spec():
    return pl.BlockSpec((D, D), lambda i: (0, 0))


def _b_spec():
    return pl.BlockSpec((1, D), lambda i: (0, 0))


def _agg_spec():
    return pl.BlockSpec((NC, ROWS_TC, DH), lambda i: (0, i, 0))


def _r_kernel(x, W, b):
    return pl.pallas_call(
        _r_body,
        grid=(GRID_TC,),
        in_specs=[_rows_spec(), _w_spec(), _b_spec()],
        out_specs=_rows_spec(),
        out_shape=jax.ShapeDtypeStruct((N, D), jnp.float32),
    )(x, W, b.reshape(1, D))


def _mid(agg, r, Wl):
    return pl.pallas_call(
        _mid_body,
        grid=(GRID_TC,),
        in_specs=[_agg_spec(), _rows_spec(), _w_spec()],
        out_specs=_rows_spec(),
        out_shape=jax.ShapeDtypeStruct((N, D), jnp.float32),
    )(agg, r, Wl)


def _fin(agg, r, Wl):
    return pl.pallas_call(
        _fin_body,
        grid=(GRID_TC,),
        in_specs=[_agg_spec(), _rows_spec(), _w_spec()],
        out_specs=_rows_spec(),
        out_shape=jax.ShapeDtypeStruct((N, D), jnp.float32),
    )(agg, r, Wl)


def kernel(x, edge_index_list, Wl0, Wr0, b0, Wl1, Wr1, b1):
    src0 = edge_index_list[0, 0].reshape(NS, NCHUNK, B)
    dst0 = edge_index_list[0, 1].reshape(NS, NCHUNK, B)
    src1, dst1 = lax.optimization_barrier(
        (edge_index_list[1, 0].reshape(NS, NCHUNK, B),
         edge_index_list[1, 1].reshape(NS, NCHUNK, B)))

    zrows = jnp.zeros((XCH, DH), jnp.float32)
    zcnt = jnp.zeros((NP,), jnp.float32)
    ones = jnp.ones((B,), jnp.float32)

    mean0 = _seg_sum_sc(x[:, :DH], x[:, DH:], src0, dst0, zrows, zcnt, ones)
    r0 = _r_kernel(x, Wr0, b0)
    h = _mid(mean0, r0, Wl0)

    mean1 = _seg_sum_sc(h[:, :DH], h[:, DH:], src1, dst1, zrows, zcnt, ones)
    r1 = _r_kernel(h, Wr1, b1)
    return _fin(mean1, r1, Wl1)


# speedup vs baseline: 1.2607x; 1.0041x over previous
import functools

import jax
import jax.numpy as jnp
from jax import lax
from jax.experimental import pallas as pl
from jax.experimental.pallas import tpu as pltpu
from jax.experimental.pallas import tpu_sc as plsc

N = 10000
E = 320000
D = 128
DH = D // 2

NC = 2
NS = 16
EPT = E // NS
B = 125
NCHUNK = EPT // B
NSLOT = 2
NR = NCHUNK // NSLOT
NP = 10240
RPT = NP // NS
XCH = 128
NX = RPT // XCH

ROWS_TC = 1000
GRID_TC = N // ROWS_TC


def _seg_sum_sc(t_lo, t_hi, src3, dst3, zrows, zcnt, ones):
    mesh = plsc.VectorSubcoreMesh(core_axis_name="c", subcore_axis_name="s")

    @functools.partial(
        pl.kernel,
        out_type=jax.ShapeDtypeStruct((NC, NP, DH), jnp.float32),
        mesh=mesh,
        scratch_types=(
            pltpu.VMEM((NCHUNK, B), jnp.int32),
            pltpu.VMEM((NCHUNK, B), jnp.int32),
            [pltpu.VMEM((B, DH), jnp.float32) for _ in range(NSLOT)],
            pltpu.VMEM((XCH, DH), jnp.float32),
            pltpu.VMEM((B,), jnp.float32),
            pltpu.VMEM((RPT,), jnp.float32),
            pltpu.VMEM_SHARED((NP, DH), jnp.float32),
            pltpu.VMEM_SHARED((NP,), jnp.float32),
            [pltpu.SemaphoreType.DMA for _ in range(NSLOT)],
            [pltpu.SemaphoreType.DMA for _ in range(NSLOT)],
        ),
        compiler_params=pltpu.CompilerParams(use_tc_tiling_on_sc=False),
    )
    def k(tlo_hbm, thi_hbm, src_hbm, dst_hbm, zrows_hbm, zcnt_hbm, ones_hbm,
          mean_out,
          src_v, dst_v, bufs, xfer, ones_v, cbuf,
          agg_sh, cnt_sh, gsems, ssems):
        c = lax.axis_index("c")
        s = lax.axis_index("s")
        base = s * RPT

        def fire_gather(j, i):
            @pl.when(c == 0)
            def _lo():
                pltpu.async_copy(tlo_hbm.at[src_v.at[j]], bufs[i], gsems[i])

            @pl.when(c == 1)
            def _hi():
                pltpu.async_copy(thi_hbm.at[src_v.at[j]], bufs[i], gsems[i])

        def wait_gather(i):
            pltpu.make_async_copy(tlo_hbm.at[src_v.at[0]], bufs[i],
                                  gsems[i]).wait()

        def do_scatter(j, i):
            pltpu.async_copy(ones_v, cnt_sh.at[dst_v.at[j]], ssems[i],
                             add=True)
            pltpu.sync_copy(bufs[i], agg_sh.at[dst_v.at[j]], add=True)

        def wait_cnt(i):
            pltpu.make_async_copy(ones_v, cnt_sh.at[dst_v.at[0]],
                                  ssems[i]).wait()

        pltpu.sync_copy(src_hbm.at[s], src_v)
        for i in range(NSLOT):
            fire_gather(i, i)
        pltpu.sync_copy(dst_hbm.at[s], dst_v)
        pltpu.sync_copy(ones_hbm, ones_v)

        pltpu.sync_copy(zrows_hbm, xfer)
        for kx in range(NX):
            pltpu.sync_copy(xfer, agg_sh.at[pl.ds(base + kx * XCH, XCH)])

        @pl.when(s == 0)
        def _zero_cnt():
            pltpu.sync_copy(zcnt_hbm, cnt_sh)

        plsc.subcore_barrier()

        @pl.loop(0, NR - 1)
        def _main(r):
            jj = r * NSLOT
            for i in range(NSLOT):
                wait_gather(i)
                do_scatter(jj + i, i)
                wait_cnt(i)
                fire_gather(jj + NSLOT + i, i)

        jj = (NR - 1) * NSLOT
        for i in range(NSLOT):
            wait_gather(i)
            do_scatter(jj + i, i)
            wait_cnt(i)

        plsc.subcore_barrier()

        pltpu.sync_copy(cnt_sh.at[pl.ds(base, RPT)], cbuf)
        for kx in range(NX):
            pltpu.sync_copy(agg_sh.at[pl.ds(base + kx * XCH, XCH)], xfer)

            @pl.loop(0, XCH // 16)
            def _rows(g):
                inv = 1.0 / jnp.maximum(
                    cbuf[pl.ds(kx * XCH + g * 16, 16)], 1.0)
                for l in range(16):
                    row = g * 16 + l
                    for col in range(DH // 16):
                        sl = pl.ds(col * 16, 16)
                        xfer[row, sl] = xfer[row, sl] * inv[l]

            pltpu.sync_copy(xfer, mean_out.at[c, pl.ds(base + kx * XCH, XCH)])

    return k(t_lo, t_hi, src3, dst3, zrows, zcnt, ones)


def _r_body(x_ref, w_ref, b_ref, r_ref):
    r_ref[...] = (
        jnp.dot(x_ref[...], w_ref[...], preferred_element_type=jnp.float32)
        + b_ref[...]
    )


def _mid_body(agg_ref, r_ref, wl_ref, h_ref):
    mean = jnp.concatenate([agg_ref[0], agg_ref[1]], axis=-1)
    h_ref[...] = jnp.maximum(
        jnp.dot(mean, wl_ref[...], preferred_element_type=jnp.float32)
        + r_ref[...],
        0.0,
    )


def _fin_body(agg_ref, r_ref, wl_ref, out_ref):
    mean = jnp.concatenate([agg_ref[0], agg_ref[1]], axis=-1)
    out_ref[...] = (
        jnp.dot(mean, wl_ref[...], preferred_element_type=jnp.float32)
        + r_ref[...]
    )


def _rows_spec():
    return pl.BlockSpec((ROWS_TC, D), lambda i: (i, 0))


def _w_spec():
    return pl.BlockSpec((D, D), lambda i: (0, 0))


def _b_spec():
    return pl.BlockSpec((1, D), lambda i: (0, 0))


def _agg_spec():
    return pl.BlockSpec((NC, ROWS_TC, DH), lambda i: (0, i, 0))


def _r_kernel(x, W, b):
    return pl.pallas_call(
        _r_body,
        grid=(GRID_TC,),
        in_specs=[_rows_spec(), _w_spec(), _b_spec()],
        out_specs=_rows_spec(),
        out_shape=jax.ShapeDtypeStruct((N, D), jnp.float32),
    )(x, W, b.reshape(1, D))


def _mid(agg, r, Wl):
    return pl.pallas_call(
        _mid_body,
        grid=(GRID_TC,),
        in_specs=[_agg_spec(), _rows_spec(), _w_spec()],
        out_specs=_rows_spec(),
        out_shape=jax.ShapeDtypeStruct((N, D), jnp.float32),
    )(agg, r, Wl)


def _fin(agg, r, Wl):
    return pl.pallas_call(
        _fin_body,
        grid=(GRID_TC,),
        in_specs=[_agg_spec(), _rows_spec(), _w_spec()],
        out_specs=_rows_spec(),
        out_shape=jax.ShapeDtypeStruct((N, D), jnp.float32),
    )(agg, r, Wl)


def kernel(x, edge_index_list, Wl0, Wr0, b0, Wl1, Wr1, b1):
    src0 = edge_index_list[0, 0].reshape(NS, NCHUNK, B)
    dst0 = edge_index_list[0, 1].reshape(NS, NCHUNK, B)
    src1, dst1 = lax.optimization_barrier(
        (edge_index_list[1, 0].reshape(NS, NCHUNK, B),
         edge_index_list[1, 1].reshape(NS, NCHUNK, B)))

    zrows = jnp.zeros((XCH, DH), jnp.float32)
    zcnt = jnp.zeros((NP,), jnp.float32)
    ones = jnp.ones((B,), jnp.float32)

    mean0 = _seg_sum_sc(x[:, :DH], x[:, DH:], src0, dst0, zrows, zcnt, ones)
    r0 = _r_kernel(x, Wr0, b0)
    h = _mid(mean0, r0, Wl0)

    mean1 = _seg_sum_sc(h[:, :DH], h[:, DH:], src1, dst1, zrows, zcnt, ones)
    r1 = _r_kernel(h, Wr1, b1)
    return _fin(mean1, r1, Wl1)
